# baseline (speedup 1.0000x reference)
import dataclasses
import functools

import jax
import jax.numpy as jnp
from jax import lax
from jax.experimental import pallas as pl
from jax.experimental.pallas import tpu as pltpu
from jax.experimental.pallas import tpu_sc as plsc

NC = 2
NS = 16
NW = NC * NS
CH = 128
BR = 1024


def _sc_mesh():
    return plsc.VectorSubcoreMesh(
        core_axis_name="c", subcore_axis_name="s", num_cores=NC, num_subcores=NS
    )


def _deg_partials(pk_p, zeros1d, npad, ept, nch):
    @functools.partial(
        pl.kernel,
        out_type=jax.ShapeDtypeStruct((NW, npad), jnp.float32),
        mesh=_sc_mesh(),
        scratch_types=[
            pltpu.VMEM((nch, CH), jnp.int32),
            pltpu.VMEM((npad,), jnp.float32),
        ],
        compiler_params=dataclasses.replace(
            pltpu.CompilerParams(), needs_layout_passes=False
        ),
    )
    def deg_kernel(pk_hbm, zeros_hbm, out_hbm, pk_all, hist):
        c = lax.axis_index("c")
        s = lax.axis_index("s")
        w = c * NS + s
        pltpu.sync_copy(zeros_hbm, hist)
        pltpu.sync_copy(pk_hbm.at[pl.ds(w * nch, nch)], pk_all)

        @pl.loop(0, nch)
        def _(j):
            @pl.loop(0, CH // 16)
            def __(k):
                idxv = pk_all.at[j][pl.ds(k * 16, 16)] >> 16
                plsc.addupdate_scatter(hist, [idxv], jnp.full((16,), 1.0, jnp.float32))

        pltpu.sync_copy(hist, out_hbm.at[w])

    return deg_kernel(pk_p, zeros1d)


def _scatter_partials(z, pk_p, zeros_row, npad, ept, nch):
    @functools.partial(
        pl.kernel,
        out_type=jax.ShapeDtypeStruct((NC, npad, 128), jnp.float32),
        mesh=_sc_mesh(),
        scratch_types=[
            pltpu.VMEM((nch, CH), jnp.int32),
            pltpu.VMEM((CH,), jnp.int32),
            pltpu.VMEM((CH,), jnp.int32),
            pltpu.VMEM((CH,), jnp.int32),
            pltpu.VMEM((CH,), jnp.int32),
            pltpu.VMEM((CH, 128), jnp.float32),
            pltpu.VMEM((CH, 128), jnp.float32),
            pltpu.VMEM_SHARED((npad, 128), jnp.float32),
            pltpu.SemaphoreType.DMA,
            pltpu.SemaphoreType.DMA,
            pltpu.SemaphoreType.DMA,
            pltpu.SemaphoreType.DMA,
        ],
        compiler_params=dataclasses.replace(
            pltpu.CompilerParams(), needs_layout_passes=False
        ),
    )
    def scat_kernel(z_hbm, pk_hbm, zeros_hbm, out_hbm,
                    pk_all, sidx0, didx0, sidx1, didx1, rows0, rows1, acc,
                    sem_g0, sem_g1, sem_s0, sem_s1):
        c = lax.axis_index("c")
        s = lax.axis_index("s")
        w = c * NS + s
        rpt = npad // NS
        pltpu.sync_copy(zeros_hbm.at[pl.ds(s * rpt, rpt)], acc.at[pl.ds(s * rpt, rpt)])
        pltpu.sync_copy(pk_hbm.at[pl.ds(w * nch, nch)], pk_all)

        def unpack(j, sidx_b, didx_b):
            @pl.loop(0, CH // 16)
            def _(k):
                v = pk_all.at[j][pl.ds(k * 16, 16)]
                sidx_b[pl.ds(k * 16, 16)] = v & 0xFFFF
                didx_b[pl.ds(k * 16, 16)] = v >> 16

        plsc.subcore_barrier()

        unpack(0, sidx0, didx0)
        unpack(1, sidx1, didx1)
        pltpu.async_copy(z_hbm.at[sidx0], rows0, sem_g0)
        pltpu.async_copy(z_hbm.at[sidx1], rows1, sem_g1)

        @pl.loop(0, nch // 2)
        def _(p):
            j0 = 2 * p
            pltpu.make_async_copy(z_hbm.at[sidx0], rows0, sem_g0).wait()
            pltpu.async_copy(rows0, acc.at[didx0], sem_s0, add=True)
            pltpu.make_async_copy(z_hbm.at[sidx1], rows1, sem_g1).wait()
            pltpu.async_copy(rows1, acc.at[didx1], sem_s1, add=True)
            pltpu.make_async_copy(rows0, acc.at[didx0], sem_s0).wait()

            @pl.when(j0 + 2 < nch)
            def _g0():
                unpack(j0 + 2, sidx0, didx0)
                pltpu.async_copy(z_hbm.at[sidx0], rows0, sem_g0)

            pltpu.make_async_copy(rows1, acc.at[didx1], sem_s1).wait()

            @pl.when(j0 + 3 < nch)
            def _g1():
                unpack(j0 + 3, sidx1, didx1)
                pltpu.async_copy(z_hbm.at[sidx1], rows1, sem_g1)

        plsc.subcore_barrier()
        pltpu.sync_copy(acc.at[pl.ds(s * rpt, rpt)], out_hbm.at[c, pl.ds(s * rpt, rpt)])

    return scat_kernel(z, pk_p, zeros_row)


def _row_spec():
    return pl.BlockSpec((BR, 128), lambda i: (i, 0))


def _w_spec():
    return pl.BlockSpec((128, 128), lambda i: (0, 0))


def _b_spec():
    return pl.BlockSpec((1, 128), lambda i: (0, 0))


def _s_spec():
    return pl.BlockSpec((2, BR, 128), lambda i: (0, i, 0))


def _dinv_and_z0(degp, xw0):
    npad = xw0.shape[0]

    def body(dp_ref, xw_ref, dv_ref, z_ref):
        dp = dp_ref[...]
        ones = jnp.ones((NW, 128), jnp.float32)
        deg = 1.0 + lax.dot_general(
            dp, ones, (((0,), (0,)), ((), ())), preferred_element_type=jnp.float32
        )
        dv = lax.rsqrt(deg)
        dv_ref[...] = dv
        z_ref[...] = dv * xw_ref[...]

    return pl.pallas_call(
        body,
        grid=(npad // BR,),
        in_specs=[pl.BlockSpec((NW, BR), lambda i: (0, i)), _row_spec()],
        out_specs=[_row_spec(), _row_spec()],
        out_shape=[
            jax.ShapeDtypeStruct((npad, 128), jnp.float32),
            jax.ShapeDtypeStruct((npad, 128), jnp.float32),
        ],
    )(degp, xw0)


def _matmul(act, w):
    npad = act.shape[0]

    def body(a_ref, w_ref, z_ref):
        z_ref[...] = jnp.dot(a_ref[...], w_ref[...], preferred_element_type=jnp.float32)

    return pl.pallas_call(
        body,
        grid=(npad // BR,),
        in_specs=[_row_spec(), _w_spec()],
        out_specs=_row_spec(),
        out_shape=jax.ShapeDtypeStruct((npad, 128), jnp.float32),
    )(act, w)


def _linear(act, w, b):
    npad = act.shape[0]

    def body(a_ref, w_ref, b_ref, o_ref):
        o_ref[...] = (
            jnp.dot(a_ref[...], w_ref[...], preferred_element_type=jnp.float32)
            + b_ref[...]
        )

    return pl.pallas_call(
        body,
        grid=(npad // BR,),
        in_specs=[_row_spec(), _w_spec(), _b_spec()],
        out_specs=_row_spec(),
        out_shape=jax.ShapeDtypeStruct((npad, 128), jnp.float32),
    )(act, w, b)


def _stage_b(s, z0, dinv, b, w):
    npad = z0.shape[0]

    def body(s_ref, z0_ref, dv_ref, b_ref, w_ref, h_ref, z1_ref):
        dv = dv_ref[...]
        g = dv * (s_ref[0] + s_ref[1] + z0_ref[...]) + b_ref[...]
        h = jnp.maximum(g, 0.0)
        h_ref[...] = h
        z1_ref[...] = dv * jnp.dot(h, w_ref[...], preferred_element_type=jnp.float32)

    return pl.pallas_call(
        body,
        grid=(npad // BR,),
        in_specs=[_s_spec(), _row_spec(), _row_spec(), _b_spec(), _w_spec()],
        out_specs=[_row_spec(), _row_spec()],
        out_shape=[
            jax.ShapeDtypeStruct((npad, 128), jnp.float32),
            jax.ShapeDtypeStruct((npad, 128), jnp.float32),
        ],
    )(s, z0, dinv, b, w)


def _stage_c(s, z1, dinv, b, skip, w):
    npad = z1.shape[0]

    def body(s_ref, z1_ref, dv_ref, b_ref, k_ref, w_ref, z2_ref):
        dv = dv_ref[...]
        g = dv * (s_ref[0] + s_ref[1] + z1_ref[...]) + b_ref[...]
        h = jnp.maximum(g, 0.0) + k_ref[...]
        z2_ref[...] = dv * jnp.dot(h, w_ref[...], preferred_element_type=jnp.float32)

    return pl.pallas_call(
        body,
        grid=(npad // BR,),
        in_specs=[_s_spec(), _row_spec(), _row_spec(), _b_spec(), _row_spec(), _w_spec()],
        out_specs=_row_spec(),
        out_shape=jax.ShapeDtypeStruct((npad, 128), jnp.float32),
    )(s, z1, dinv, b, skip, w)


def _stage_d(s, z2, dinv, b):
    npad = z2.shape[0]

    def body(s_ref, z2_ref, dv_ref, b_ref, o_ref):
        o_ref[...] = dv_ref[...] * (s_ref[0] + s_ref[1] + z2_ref[...]) + b_ref[...]

    return pl.pallas_call(
        body,
        grid=(npad // BR,),
        in_specs=[_s_spec(), _row_spec(), _row_spec(), _b_spec()],
        out_specs=_row_spec(),
        out_shape=jax.ShapeDtypeStruct((npad, 128), jnp.float32),
    )(s, z2, dinv, b)


def kernel(x, edge_index, W0, b0, W1, b1, W2, b2, Ws, bs):
    n, d_in = x.shape
    e = edge_index.shape[1]
    assert d_in == 128

    npad = -(-(n + 1) // BR) * BR
    ept = -(-e // (NW * 8 * CH)) * 8 * CH
    epad = ept * NW
    nch = ept // CH

    src = edge_index[0]
    dst = edge_index[1]
    pad_e = epad - e
    src_p = jnp.concatenate([src, jnp.zeros((pad_e,), jnp.int32)])
    dst_p = jnp.concatenate([dst, jnp.full((pad_e,), n, jnp.int32)])
    pk_p = (src_p | (dst_p << 16)).reshape(-1, CH)
    x_p = jnp.zeros((npad, d_in), jnp.float32).at[:n].set(x)
    zeros1d = jnp.zeros((npad,), jnp.float32)
    zeros_row = jnp.zeros((npad, 128), jnp.float32)
    b0r = b0.reshape(1, 128)
    b1r = b1.reshape(1, 128)
    b2r = b2.reshape(1, 128)
    bsr = bs.reshape(1, 128)

    degp = _deg_partials(pk_p, zeros1d, npad, ept, nch)
    xw0 = _matmul(x_p, W0)
    dinv, z0 = _dinv_and_z0(degp, xw0)
    s0 = _scatter_partials(z0, pk_p, zeros_row, npad, ept, nch)
    h0, z1 = _stage_b(s0, z0, dinv, b0r, W1)
    skip = _linear(h0, Ws, bsr)
    s1 = _scatter_partials(z1, pk_p, zeros_row, npad, ept, nch)
    z2 = _stage_c(s1, z1, dinv, b1r, skip, W2)
    s2 = _scatter_partials(z2, pk_p, zeros_row, npad, ept, nch)
    out = _stage_d(s2, z2, dinv, b2r)
    return out[:n]
